# baseline (speedup 1.0000x reference)
import functools

import jax
import jax.numpy as jnp
from jax import lax
from jax.experimental import pallas as pl
from jax.experimental.pallas import tpu as pltpu
from jax.experimental.pallas import tpu_sc as plsc

INPUT_DIM = 3
INDEX_WIDTH = 100
FEATURE_SIZE = 64
BATCH = 16384
ROWS = INDEX_WIDTH ** INPUT_DIM

_INFO = plsc.get_sparse_core_info()
_NC, _NS, _L = _INFO.num_cores, _INFO.num_subcores, _INFO.num_lanes
_NW = _NC * _NS

_SC_ROWS = 6656
_TC_ROWS = BATCH - _SC_ROWS
_BPW = _SC_ROWS // _NW

_TC_BR = 256
_TC_NB = _TC_ROWS // _TC_BR
_TC_SEMS = 8


def _sc_body(x0_hbm, x1_hbm, x2_hbm, table_hbm, out_hbm,
             c0, c1, c2, rows_v, sem):
    wid = lax.axis_index("s") * _NC + lax.axis_index("c")
    base = wid * _BPW

    pltpu.sync_copy(x0_hbm.at[pl.ds(base, _BPW)], c0)
    pltpu.sync_copy(x1_hbm.at[pl.ds(base, _BPW)], c1)
    pltpu.sync_copy(x2_hbm.at[pl.ds(base, _BPW)], c2)

    scale = jnp.float32(INDEX_WIDTH)
    iota = lax.iota(jnp.int32, _L)
    zero16 = jnp.zeros((_L,), jnp.int32)

    def fire(i, carry):
        s = pl.ds(i * _L, _L)
        lin = (c0[s] * scale).astype(jnp.int32)
        lin += (c1[s] * scale).astype(jnp.int32) * INDEX_WIDTH
        lin += (c2[s] * scale).astype(jnp.int32) * (INDEX_WIDTH * INDEX_WIDTH)
        for l in range(_L):
            r = jnp.sum(jnp.where(iota == l, lin, zero16))
            pltpu.make_async_copy(
                table_hbm.at[pl.ds(r, 1), :],
                rows_v.at[pl.ds(i * _L + l, 1), :],
                sem,
            ).start()
        return carry

    lax.fori_loop(0, _BPW // _L, fire, 0)

    pltpu.make_async_copy(
        table_hbm.at[pl.ds(0, _BPW), :], rows_v, sem
    ).wait()

    pltpu.sync_copy(rows_v, out_hbm.at[pl.ds(base, _BPW), :])


def _sc_gather(x0, x1, x2, table):
    mesh = plsc.VectorSubcoreMesh(core_axis_name="c", subcore_axis_name="s")
    return pl.kernel(
        _sc_body,
        out_type=jax.ShapeDtypeStruct((_SC_ROWS, FEATURE_SIZE), jnp.float32),
        mesh=mesh,
        scratch_types=[
            pltpu.VMEM((_BPW,), jnp.float32),
            pltpu.VMEM((_BPW,), jnp.float32),
            pltpu.VMEM((_BPW,), jnp.float32),
            pltpu.VMEM((_BPW, FEATURE_SIZE), jnp.float32),
            pltpu.SemaphoreType.DMA,
        ],
        compiler_params=pltpu.CompilerParams(
            needs_layout_passes=False,
            skip_device_barrier=True,
            has_side_effects=pltpu.SideEffectType.DATAFLOW_SIDE_EFFECTING,
        ),
    )(x0, x1, x2, table)


def _tc_body(idx_ref, table_hbm, out_blk, rows_v, sems):
    b = pl.program_id(0)

    def fire(j, carry):
        r = idx_ref[b * _TC_BR + j]
        pltpu.make_async_copy(
            table_hbm.at[pl.ds(r, 1), :],
            rows_v.at[pl.ds(j, 1), :],
            sems.at[lax.rem(j, _TC_SEMS)],
        ).start()
        return carry

    lax.fori_loop(0, _TC_BR, fire, 0)

    n = _TC_BR // _TC_SEMS
    for s in range(_TC_SEMS):
        pltpu.make_async_copy(
            table_hbm.at[pl.ds(0, n), :], rows_v.at[pl.ds(0, n), :],
            sems.at[s],
        ).wait()

    out_blk[...] = rows_v[...]


def _tc_gather(lin_tc, table):
    grid_spec = pltpu.PrefetchScalarGridSpec(
        num_scalar_prefetch=1,
        grid=(_TC_NB,),
        in_specs=[pl.BlockSpec(memory_space=pl.ANY)],
        out_specs=pl.BlockSpec((_TC_BR, FEATURE_SIZE), lambda i, idx: (i, 0)),
        scratch_shapes=[
            pltpu.VMEM((_TC_BR, FEATURE_SIZE), jnp.float32),
            pltpu.SemaphoreType.DMA((_TC_SEMS,)),
        ],
    )
    return pl.pallas_call(
        _tc_body,
        grid_spec=grid_spec,
        out_shape=jax.ShapeDtypeStruct((_TC_ROWS, FEATURE_SIZE),
                                       jnp.float32),
    )(lin_tc, table)


@jax.jit
def _lookup(x0, x1, x2, lin_tc, table):
    sc_out = _sc_gather(x0, x1, x2, table)
    tc_out = _tc_gather(lin_tc, table)
    return jnp.concatenate([sc_out, tc_out], axis=0)


def kernel(indices, table):
    x0 = indices[:_SC_ROWS, 0]
    x1 = indices[:_SC_ROWS, 1]
    x2 = indices[:_SC_ROWS, 2]
    tail = indices[_SC_ROWS:]
    scaled = (tail * jnp.float32(INDEX_WIDTH)).astype(jnp.int32)
    lin_tc = (scaled[:, 0] + scaled[:, 1] * INDEX_WIDTH
              + scaled[:, 2] * (INDEX_WIDTH * INDEX_WIDTH))
    return _lookup(x0, x1, x2, lin_tc, table)


# speedup vs baseline: 1.2826x; 1.2826x over previous
import functools

import jax
import jax.numpy as jnp
from jax import lax
from jax.experimental import pallas as pl
from jax.experimental.pallas import tpu as pltpu
from jax.experimental.pallas import tpu_sc as plsc

INPUT_DIM = 3
INDEX_WIDTH = 100
FEATURE_SIZE = 64
BATCH = 16384
ROWS = INDEX_WIDTH ** INPUT_DIM

_INFO = plsc.get_sparse_core_info()
_NC, _NS, _L = _INFO.num_cores, _INFO.num_subcores, _INFO.num_lanes
_NW = _NC * _NS
_BPW = BATCH // _NW


def _lookup_body(x0_hbm, x1_hbm, x2_hbm, table_hbm, out_hbm,
                 c0, c1, c2, rows_v, sem):
    wid = lax.axis_index("s") * _NC + lax.axis_index("c")
    base = wid * _BPW

    pltpu.sync_copy(x0_hbm.at[pl.ds(base, _BPW)], c0)
    pltpu.sync_copy(x1_hbm.at[pl.ds(base, _BPW)], c1)
    pltpu.sync_copy(x2_hbm.at[pl.ds(base, _BPW)], c2)

    scale = jnp.float32(INDEX_WIDTH)
    iota = lax.iota(jnp.int32, _L)
    zero16 = jnp.zeros((_L,), jnp.int32)

    def fire(i, carry):
        s = pl.ds(i * _L, _L)
        lin = (c0[s] * scale).astype(jnp.int32)
        lin += (c1[s] * scale).astype(jnp.int32) * INDEX_WIDTH
        lin += (c2[s] * scale).astype(jnp.int32) * (INDEX_WIDTH * INDEX_WIDTH)
        for l in range(_L):
            r = jnp.sum(jnp.where(iota == l, lin, zero16))
            pltpu.make_async_copy(
                table_hbm.at[pl.ds(r, 1), :],
                rows_v.at[pl.ds(i * _L + l, 1), :],
                sem,
            ).start()
        return carry

    lax.fori_loop(0, _BPW // _L, fire, 0)

    pltpu.make_async_copy(
        table_hbm.at[pl.ds(0, _BPW), :], rows_v, sem
    ).wait()

    pltpu.sync_copy(rows_v, out_hbm.at[pl.ds(base, _BPW), :])


@jax.jit
def _lookup(x0, x1, x2, table):
    mesh = plsc.VectorSubcoreMesh(core_axis_name="c", subcore_axis_name="s")
    return pl.kernel(
        _lookup_body,
        out_type=jax.ShapeDtypeStruct((BATCH, FEATURE_SIZE), jnp.float32),
        mesh=mesh,
        scratch_types=[
            pltpu.VMEM((_BPW,), jnp.float32),
            pltpu.VMEM((_BPW,), jnp.float32),
            pltpu.VMEM((_BPW,), jnp.float32),
            pltpu.VMEM((_BPW, FEATURE_SIZE), jnp.float32),
            pltpu.SemaphoreType.DMA,
        ],
        compiler_params=pltpu.CompilerParams(needs_layout_passes=False),
    )(x0, x1, x2, table)


def kernel(indices, table):
    return _lookup(indices[:, 0], indices[:, 1], indices[:, 2], table)
